# baseline (speedup 1.0000x reference)
import functools

import jax
import jax.numpy as jnp
from jax import lax
from jax.experimental import pallas as pl
from jax.experimental.pallas import tpu as pltpu
from jax.experimental.pallas import tpu_sc as plsc

N = 16384
DIN = 1025
DPADIN = 1152
DOUT = 1024
BLK = 256
NB = N // BLK + 5
NPAD = NB * BLK
NWORK = 32


def _route(x):
    i32 = jnp.int32
    t = x[:, 0]
    b = ((t >= 0.2).astype(i32) + (t >= 0.4).astype(i32)
         + (t >= 0.6).astype(i32) + (t >= 0.8).astype(i32))
    oh = (b[:, None] == jnp.arange(5, dtype=i32)[None, :]).astype(i32)
    csum = jnp.cumsum(oh, axis=0)
    counts = csum[-1]
    rank = jnp.sum(oh * csum, axis=1) - 1
    blocks_e = (counts + BLK - 1) // BLK
    bstart = jnp.concatenate([jnp.zeros(1, i32), jnp.cumsum(blocks_e).astype(i32)])
    pos = bstart[b] * BLK + rank
    gids = jnp.arange(NB, dtype=i32)
    block_expert = ((gids >= bstart[1]).astype(i32) + (gids >= bstart[2]).astype(i32)
                    + (gids >= bstart[3]).astype(i32) + (gids >= bstart[4]).astype(i32))
    return pos, block_expert


def _sc_row_scatter(table, pos3, n_out, chunk, nbuf):
    d = table.shape[1]
    nch = pos3.shape[1]
    per_w = nch * chunk
    mesh = plsc.VectorSubcoreMesh(core_axis_name="c", subcore_axis_name="s")

    @functools.partial(
        pl.kernel,
        out_type=jax.ShapeDtypeStruct((n_out, d), table.dtype),
        mesh=mesh,
        scratch_types=(
            [pltpu.VMEM((nch, chunk), jnp.int32)]
            + [pltpu.VMEM((chunk, d), table.dtype) for _ in range(nbuf)]
            + [pltpu.SemaphoreType.DMA for _ in range(2 * nbuf)]
        ),
    )
    def sk(table_hbm, pos_hbm, out_hbm, idx_v, *rest):
        bufs = rest[:nbuf]
        semr = rest[nbuf:2 * nbuf]
        semw = rest[2 * nbuf:]
        wid = lax.axis_index("s") * 2 + lax.axis_index("c")
        base = wid * per_w
        pltpu.sync_copy(pos_hbm.at[wid], idx_v)

        r = [None] * nch
        w = [None] * nch

        def start_read(j):
            r[j] = pltpu.async_copy(
                table_hbm.at[pl.ds(base + j * chunk, chunk)],
                bufs[j % nbuf], semr[j % nbuf])

        start_read(0)
        if nch > 1:
            start_read(1)
        for i in range(nch):
            r[i].wait()
            w[i] = pltpu.async_copy(
                bufs[i % nbuf], out_hbm.at[idx_v.at[i]], semw[i % nbuf])
            j = i + 2
            if j < nch:
                if j >= nbuf:
                    w[j - nbuf].wait()
                start_read(j)
        for i in range(max(0, nch - nbuf), nch):
            w[i].wait()

    return sk(table, pos3)


def _sc_row_gather(table, idx, n_rows, chunk, nbuf):
    d = table.shape[1]
    per_w = n_rows // NWORK
    nch = per_w // chunk
    mesh = plsc.VectorSubcoreMesh(core_axis_name="c", subcore_axis_name="s")

    @functools.partial(
        pl.kernel,
        out_type=jax.ShapeDtypeStruct((n_rows, d), table.dtype),
        mesh=mesh,
        scratch_types=(
            [pltpu.VMEM((per_w,), jnp.int32)]
            + [pltpu.VMEM((chunk, d), table.dtype) for _ in range(nbuf)]
            + [pltpu.SemaphoreType.DMA for _ in range(2 * nbuf)]
        ),
    )
    def gk(table_hbm, idx_hbm, out_hbm, idx_v, *rest):
        bufs = rest[:nbuf]
        semg = rest[nbuf:2 * nbuf]
        semw = rest[2 * nbuf:]
        wid = lax.axis_index("s") * 2 + lax.axis_index("c")
        base = wid * per_w
        pltpu.sync_copy(idx_hbm.at[pl.ds(base, per_w)], idx_v)

        g = [None] * nch
        w = [None] * nch

        def start_gather(j):
            g[j] = pltpu.async_copy(
                table_hbm.at[idx_v.at[pl.ds(j * chunk, chunk)]],
                bufs[j % nbuf], semg[j % nbuf])

        start_gather(0)
        if nch > 1:
            start_gather(1)
        for i in range(nch):
            g[i].wait()
            w[i] = pltpu.async_copy(
                bufs[i % nbuf], out_hbm.at[pl.ds(base + i * chunk, chunk)],
                semw[i % nbuf])
            j = i + 2
            if j < nch:
                if j >= nbuf:
                    w[j - nbuf].wait()
                start_gather(j)
        for i in range(max(0, nch - nbuf), nch):
            w[i].wait()

    return gk(table, idx)


def _mlp_body(be_ref, x_ref, w0_ref, b0_ref, t0_ref, w1_ref, b1_ref, t1_ref,
              w2_ref, b2_ref, t2_ref, o_ref):
    xb = x_ref[...]
    t = xb[:, 1024:1025]
    h = jnp.dot(xb[:, 0:1024].astype(jnp.bfloat16), w0_ref[0],
                preferred_element_type=jnp.float32)
    h = jax.nn.relu(h + t * t0_ref[0] + b0_ref[0])
    h = jnp.dot(h.astype(jnp.bfloat16), w1_ref[0],
                preferred_element_type=jnp.float32)
    h = jax.nn.relu(h + t * t1_ref[0] + b1_ref[0])
    h = jnp.dot(h.astype(jnp.bfloat16), w2_ref[0],
                preferred_element_type=jnp.float32)
    o_ref[...] = h + t * t2_ref[0] + b2_ref[0]


def _grouped_mlp(x_pad, block_expert, W0, b0, tw0, W1, b1, tw1, W2, b2, tw2):
    grid_spec = pltpu.PrefetchScalarGridSpec(
        num_scalar_prefetch=1,
        grid=(NB,),
        in_specs=[
            pl.BlockSpec((BLK, DPADIN), lambda g, be: (g, 0)),
            pl.BlockSpec((1, 1024, 2048), lambda g, be: (be[g], 0, 0)),
            pl.BlockSpec((1, 1, 2048), lambda g, be: (be[g], 0, 0)),
            pl.BlockSpec((1, 1, 2048), lambda g, be: (be[g], 0, 0)),
            pl.BlockSpec((1, 2048, 2048), lambda g, be: (be[g], 0, 0)),
            pl.BlockSpec((1, 1, 2048), lambda g, be: (be[g], 0, 0)),
            pl.BlockSpec((1, 1, 2048), lambda g, be: (be[g], 0, 0)),
            pl.BlockSpec((1, 2048, 1024), lambda g, be: (be[g], 0, 0)),
            pl.BlockSpec((1, 1, 1024), lambda g, be: (be[g], 0, 0)),
            pl.BlockSpec((1, 1, 1024), lambda g, be: (be[g], 0, 0)),
        ],
        out_specs=pl.BlockSpec((BLK, DOUT), lambda g, be: (g, 0)),
    )
    return pl.pallas_call(
        _mlp_body,
        grid_spec=grid_spec,
        out_shape=jax.ShapeDtypeStruct((NPAD, DOUT), jnp.float32),
        compiler_params=pltpu.CompilerParams(
            dimension_semantics=("arbitrary",),
        ),
    )(block_expert, x_pad, W0, b0, tw0, W1, b1, tw1, W2, b2, tw2)


def kernel(x, W0, b0, tw0, W1, b1, tw1, W2, b2, tw2):
    bf16 = jnp.bfloat16
    pos, block_expert = _route(x)
    xa = jnp.concatenate(
        [x[:, 1:], x[:, 0:1], jnp.zeros((N, DPADIN - DIN), x.dtype)], axis=1)
    pos3 = pos.reshape(NWORK, 16, 32)
    x_pad = _sc_row_scatter(xa, pos3, NPAD, chunk=32, nbuf=3)
    y_pad = _grouped_mlp(x_pad, block_expert,
                         W0.astype(bf16), b0.reshape(5, 1, 2048), tw0,
                         W1.astype(bf16), b1.reshape(5, 1, 2048), tw1,
                         W2.astype(bf16), b2.reshape(5, 1, 1024), tw2)
    return _sc_row_gather(y_pad, pos, N, chunk=32, nbuf=3)


# speedup vs baseline: 1.1331x; 1.1331x over previous
import functools

import jax
import jax.numpy as jnp
from jax import lax
from jax.experimental import pallas as pl
from jax.experimental.pallas import tpu as pltpu
from jax.experimental.pallas import tpu_sc as plsc

N = 16384
DIN = 1025
DPADIN = 1152
DOUT = 1024
BLK = 256
NB = N // BLK + 5
NPAD = NB * BLK
NWORK = 32


def _route(x):
    i32 = jnp.int32
    t = x[:, 0]
    b = ((t >= 0.2).astype(i32) + (t >= 0.4).astype(i32)
         + (t >= 0.6).astype(i32) + (t >= 0.8).astype(i32))
    oh = (b[:, None] == jnp.arange(5, dtype=i32)[None, :]).astype(i32)
    csum = jnp.cumsum(oh, axis=0)
    counts = csum[-1]
    rank = jnp.sum(oh * csum, axis=1) - 1
    blocks_e = (counts + BLK - 1) // BLK
    bstart = jnp.concatenate([jnp.zeros(1, i32), jnp.cumsum(blocks_e).astype(i32)])
    pos = bstart[b] * BLK + rank
    gids = jnp.arange(NB, dtype=i32)
    block_expert = ((gids >= bstart[1]).astype(i32) + (gids >= bstart[2]).astype(i32)
                    + (gids >= bstart[3]).astype(i32) + (gids >= bstart[4]).astype(i32))
    return pos, jnp.concatenate([block_expert, bstart[5:6]])


def _sc_row_scatter(table, pos3, n_out, chunk, nbuf):
    d = table.shape[1]
    nch = pos3.shape[1]
    per_w = nch * chunk
    mesh = plsc.VectorSubcoreMesh(core_axis_name="c", subcore_axis_name="s")

    @functools.partial(
        pl.kernel,
        out_type=jax.ShapeDtypeStruct((n_out, d), table.dtype),
        mesh=mesh,
        scratch_types=(
            [pltpu.VMEM((nch, chunk), jnp.int32)]
            + [pltpu.VMEM((chunk, d), table.dtype) for _ in range(nbuf)]
            + [pltpu.SemaphoreType.DMA for _ in range(2 * nbuf)]
        ),
    )
    def sk(table_hbm, pos_hbm, out_hbm, idx_v, *rest):
        bufs = rest[:nbuf]
        semr = rest[nbuf:2 * nbuf]
        semw = rest[2 * nbuf:]
        wid = lax.axis_index("s") * 2 + lax.axis_index("c")
        base = wid * per_w
        pltpu.sync_copy(pos_hbm.at[wid], idx_v)

        r = [None] * nch
        w = [None] * nch

        def start_read(j):
            r[j] = pltpu.async_copy(
                table_hbm.at[pl.ds(base + j * chunk, chunk)],
                bufs[j % nbuf], semr[j % nbuf])

        start_read(0)
        if nch > 1:
            start_read(1)
        for i in range(nch):
            r[i].wait()
            w[i] = pltpu.async_copy(
                bufs[i % nbuf], out_hbm.at[idx_v.at[i]], semw[i % nbuf])
            j = i + 2
            if j < nch:
                if j >= nbuf:
                    w[j - nbuf].wait()
                start_read(j)
        for i in range(max(0, nch - nbuf), nch):
            w[i].wait()

    return sk(table, pos3)


def _sc_row_gather(table, idx, n_rows, chunk, nbuf):
    d = table.shape[1]
    per_w = n_rows // NWORK
    nch = per_w // chunk
    mesh = plsc.VectorSubcoreMesh(core_axis_name="c", subcore_axis_name="s")

    @functools.partial(
        pl.kernel,
        out_type=jax.ShapeDtypeStruct((n_rows, d), table.dtype),
        mesh=mesh,
        scratch_types=(
            [pltpu.VMEM((per_w,), jnp.int32)]
            + [pltpu.VMEM((chunk, d), table.dtype) for _ in range(nbuf)]
            + [pltpu.SemaphoreType.DMA for _ in range(2 * nbuf)]
        ),
    )
    def gk(table_hbm, idx_hbm, out_hbm, idx_v, *rest):
        bufs = rest[:nbuf]
        semg = rest[nbuf:2 * nbuf]
        semw = rest[2 * nbuf:]
        wid = lax.axis_index("s") * 2 + lax.axis_index("c")
        base = wid * per_w
        pltpu.sync_copy(idx_hbm.at[pl.ds(base, per_w)], idx_v)

        g = [None] * nch
        w = [None] * nch

        def start_gather(j):
            g[j] = pltpu.async_copy(
                table_hbm.at[idx_v.at[pl.ds(j * chunk, chunk)]],
                bufs[j % nbuf], semg[j % nbuf])

        start_gather(0)
        if nch > 1:
            start_gather(1)
        for i in range(nch):
            g[i].wait()
            w[i] = pltpu.async_copy(
                bufs[i % nbuf], out_hbm.at[pl.ds(base + i * chunk, chunk)],
                semw[i % nbuf])
            j = i + 2
            if j < nch:
                if j >= nbuf:
                    w[j - nbuf].wait()
                start_gather(j)
        for i in range(max(0, nch - nbuf), nch):
            w[i].wait()

    return gk(table, idx)


def _mlp_body(be_ref, x_ref, w0_ref, b0_ref, t0_ref, w1_ref, b1_ref, t1_ref,
              w2_ref, b2_ref, t2_ref, o_ref):
    @pl.when(pl.program_id(0) < be_ref[NB])
    def _():
        xb = x_ref[...]
        t = xb[:, 0:1]
        h = jnp.dot(xb[:, 1:1025].astype(jnp.bfloat16), w0_ref[0],
                    preferred_element_type=jnp.float32)
        h = jax.nn.relu(h + t * t0_ref[0] + b0_ref[0])
        h = jnp.dot(h.astype(jnp.bfloat16), w1_ref[0],
                    preferred_element_type=jnp.float32)
        h = jax.nn.relu(h + t * t1_ref[0] + b1_ref[0])
        h = jnp.dot(h.astype(jnp.bfloat16), w2_ref[0],
                    preferred_element_type=jnp.float32)
        o_ref[...] = h + t * t2_ref[0] + b2_ref[0]


def _grouped_mlp(x_pad, block_expert, W0, b0, tw0, W1, b1, tw1, W2, b2, tw2):
    grid_spec = pltpu.PrefetchScalarGridSpec(
        num_scalar_prefetch=1,
        grid=(NB,),
        in_specs=[
            pl.BlockSpec((BLK, DPADIN), lambda g, be: (g, 0)),
            pl.BlockSpec((1, 1024, 2048), lambda g, be: (be[g], 0, 0)),
            pl.BlockSpec((1, 1, 2048), lambda g, be: (be[g], 0, 0)),
            pl.BlockSpec((1, 1, 2048), lambda g, be: (be[g], 0, 0)),
            pl.BlockSpec((1, 2048, 2048), lambda g, be: (be[g], 0, 0)),
            pl.BlockSpec((1, 1, 2048), lambda g, be: (be[g], 0, 0)),
            pl.BlockSpec((1, 1, 2048), lambda g, be: (be[g], 0, 0)),
            pl.BlockSpec((1, 2048, 1024), lambda g, be: (be[g], 0, 0)),
            pl.BlockSpec((1, 1, 1024), lambda g, be: (be[g], 0, 0)),
            pl.BlockSpec((1, 1, 1024), lambda g, be: (be[g], 0, 0)),
        ],
        out_specs=pl.BlockSpec((BLK, DOUT), lambda g, be: (g, 0)),
    )
    return pl.pallas_call(
        _mlp_body,
        grid_spec=grid_spec,
        out_shape=jax.ShapeDtypeStruct((NPAD, DOUT), jnp.float32),
        compiler_params=pltpu.CompilerParams(
            dimension_semantics=("arbitrary",),
        ),
    )(block_expert, x_pad, W0, b0, tw0, W1, b1, tw1, W2, b2, tw2)


def kernel(x, W0, b0, tw0, W1, b1, tw1, W2, b2, tw2):
    bf16 = jnp.bfloat16
    pos, block_expert = _route(x)
    xa = jnp.pad(x, ((0, 0), (0, DPADIN - DIN)))
    pos3 = pos.reshape(NWORK, 16, 32)
    x_pad = _sc_row_scatter(xa, pos3, NPAD, chunk=32, nbuf=3)
    y_pad = _grouped_mlp(x_pad, block_expert,
                         W0.astype(bf16), b0.reshape(5, 1, 2048), tw0,
                         W1.astype(bf16), b1.reshape(5, 1, 2048), tw1,
                         W2.astype(bf16), b2.reshape(5, 1, 1024), tw2)
    return _sc_row_gather(y_pad, pos, N, chunk=32, nbuf=3)
